# baseline (speedup 1.0000x reference)
import jax
import jax.numpy as jnp
from jax import lax
from jax.experimental import pallas as pl

N_TOKENS = 8192
D_MODEL = 2048
NUM_EXPERTS = 16
TOP_K = 2

N_STREAMS = 2
BLOCK_T = 512
HALF = BLOCK_T // N_STREAMS

_NT = (((1,), (1,)), ((), ()))


def _route_stream(xb, w_t, b_col, eps_blk):
    both_t = lax.dot_general(w_t, xb, _NT,
                             preferred_element_type=jnp.float32)
    both_t = both_t + b_col
    logits_t = both_t[:NUM_EXPERTS, :]
    nlogits_t = both_t[NUM_EXPERTS:, :]
    noisy = logits_t + eps_blk * jax.nn.softplus(nlogits_t)

    iota = lax.broadcasted_iota(jnp.int32, noisy.shape, 0)
    m1 = jnp.max(noisy, axis=0, keepdims=True)
    i1 = jnp.min(jnp.where(noisy == m1, iota, NUM_EXPERTS), axis=0,
                 keepdims=True)
    masked = jnp.where(iota == i1, -jnp.inf, noisy)
    m2 = jnp.max(masked, axis=0, keepdims=True)
    i2 = jnp.min(jnp.where(masked == m2, iota, NUM_EXPERTS), axis=0,
                 keepdims=True)
    keep = (iota == i1) | (iota == i2)
    z = jnp.where(keep, jnp.exp(noisy - m1), 0.0)
    out = z / jnp.sum(z, axis=0, keepdims=True)
    idx = jnp.concatenate([i1, i2], axis=0)
    return out, idx


def _router_body(*refs):
    x_refs = refs[:N_STREAMS]
    wr_ref, wn_ref, br_ref, bn_ref, eps_ref, out_ref, idx_ref = \
        refs[N_STREAMS:]
    w_t = jnp.concatenate([wr_ref[...], wn_ref[...]], axis=0)
    b_col = jnp.concatenate([br_ref[...], bn_ref[...]], axis=1).T
    for s in range(N_STREAMS):
        lo = s * HALF
        out_s, idx_s = _route_stream(x_refs[s][...], w_t, b_col,
                                     eps_ref[:, pl.ds(lo, HALF)])
        out_ref[:, pl.ds(lo, HALF)] = out_s
        idx_ref[:, pl.ds(lo, HALF)] = idx_s


def _x_spec(s):
    return pl.BlockSpec((HALF, D_MODEL), lambda i: (N_STREAMS * i + s, 0))


def kernel(x, W_route, b_route, W_noise, b_noise, eps):
    n_blocks = N_TOKENS // BLOCK_T
    wr_t = W_route.T
    wn_t = W_noise.T
    br = b_route.reshape(1, NUM_EXPERTS)
    bn = b_noise.reshape(1, NUM_EXPERTS)
    eps_t = eps.T
    out_shapes = (
        jax.ShapeDtypeStruct((NUM_EXPERTS, N_TOKENS), jnp.float32),
        jax.ShapeDtypeStruct((TOP_K, N_TOKENS), jnp.int32),
    )
    out_t, idx_t = pl.pallas_call(
        _router_body,
        grid=(n_blocks,),
        in_specs=[_x_spec(s) for s in range(N_STREAMS)] + [
            pl.BlockSpec((NUM_EXPERTS, D_MODEL), lambda i: (0, 0)),
            pl.BlockSpec((NUM_EXPERTS, D_MODEL), lambda i: (0, 0)),
            pl.BlockSpec((1, NUM_EXPERTS), lambda i: (0, 0)),
            pl.BlockSpec((1, NUM_EXPERTS), lambda i: (0, 0)),
            pl.BlockSpec((NUM_EXPERTS, BLOCK_T), lambda i: (0, i)),
        ],
        out_specs=(
            pl.BlockSpec((NUM_EXPERTS, BLOCK_T), lambda i: (0, i)),
            pl.BlockSpec((TOP_K, BLOCK_T), lambda i: (0, i)),
        ),
        out_shape=out_shapes,
    )(*([x] * N_STREAMS), wr_t, wn_t, br, bn, eps_t)
    return (out_t.T, idx_t.T)


# speedup vs baseline: 1.1044x; 1.1044x over previous
import jax
import jax.numpy as jnp
from jax import lax
from jax.experimental import pallas as pl

N_TOKENS = 8192
D_MODEL = 2048
NUM_EXPERTS = 16
TOP_K = 2

N_STREAMS = 8
BLOCK_T = 1024
HALF = BLOCK_T // N_STREAMS

_NT = (((1,), (1,)), ((), ()))


def _route_stream(xb, w_t, b_col, eps_blk):
    both_t = lax.dot_general(w_t, xb, _NT,
                             preferred_element_type=jnp.float32)
    both_t = both_t + b_col
    logits_t = both_t[:NUM_EXPERTS, :]
    nlogits_t = both_t[NUM_EXPERTS:, :]
    noisy = logits_t + eps_blk * jax.nn.softplus(nlogits_t)

    iota = lax.broadcasted_iota(jnp.int32, noisy.shape, 0)
    m1 = jnp.max(noisy, axis=0, keepdims=True)
    i1 = jnp.min(jnp.where(noisy == m1, iota, NUM_EXPERTS), axis=0,
                 keepdims=True)
    masked = jnp.where(iota == i1, -jnp.inf, noisy)
    m2 = jnp.max(masked, axis=0, keepdims=True)
    i2 = jnp.min(jnp.where(masked == m2, iota, NUM_EXPERTS), axis=0,
                 keepdims=True)
    keep = (iota == i1) | (iota == i2)
    z = jnp.where(keep, jnp.exp(noisy - m1), 0.0)
    out = z / jnp.sum(z, axis=0, keepdims=True)
    idx = jnp.concatenate([i1, i2], axis=0)
    return out, idx


def _router_body(*refs):
    x_refs = refs[:N_STREAMS]
    wr_ref, wn_ref, br_ref, bn_ref, eps_ref, out_ref, idx_ref = \
        refs[N_STREAMS:]
    w_t = jnp.concatenate([wr_ref[...], wn_ref[...]], axis=0)
    b_col = jnp.concatenate([br_ref[...], bn_ref[...]], axis=1).T
    for s in range(N_STREAMS):
        lo = s * HALF
        out_s, idx_s = _route_stream(x_refs[s][...], w_t, b_col,
                                     eps_ref[:, pl.ds(lo, HALF)])
        out_ref[:, pl.ds(lo, HALF)] = out_s
        idx_ref[:, pl.ds(lo, HALF)] = idx_s


def _x_spec(s):
    return pl.BlockSpec((HALF, D_MODEL), lambda i: (N_STREAMS * i + s, 0))


def kernel(x, W_route, b_route, W_noise, b_noise, eps):
    n_blocks = N_TOKENS // BLOCK_T
    wr_t = W_route.T
    wn_t = W_noise.T
    br = b_route.reshape(1, NUM_EXPERTS)
    bn = b_noise.reshape(1, NUM_EXPERTS)
    eps_t = eps.T
    out_shapes = (
        jax.ShapeDtypeStruct((NUM_EXPERTS, N_TOKENS), jnp.float32),
        jax.ShapeDtypeStruct((TOP_K, N_TOKENS), jnp.int32),
    )
    out_t, idx_t = pl.pallas_call(
        _router_body,
        grid=(n_blocks,),
        in_specs=[_x_spec(s) for s in range(N_STREAMS)] + [
            pl.BlockSpec((NUM_EXPERTS, D_MODEL), lambda i: (0, 0)),
            pl.BlockSpec((NUM_EXPERTS, D_MODEL), lambda i: (0, 0)),
            pl.BlockSpec((1, NUM_EXPERTS), lambda i: (0, 0)),
            pl.BlockSpec((1, NUM_EXPERTS), lambda i: (0, 0)),
            pl.BlockSpec((NUM_EXPERTS, BLOCK_T), lambda i: (0, i)),
        ],
        out_specs=(
            pl.BlockSpec((NUM_EXPERTS, BLOCK_T), lambda i: (0, i)),
            pl.BlockSpec((TOP_K, BLOCK_T), lambda i: (0, i)),
        ),
        out_shape=out_shapes,
    )(*([x] * N_STREAMS), wr_t, wn_t, br, bn, eps_t)
    return (out_t.T, idx_t.T)


# speedup vs baseline: 1.1271x; 1.0205x over previous
import jax
import jax.numpy as jnp
from jax import lax
from jax.experimental import pallas as pl

N_TOKENS = 8192
D_MODEL = 2048
NUM_EXPERTS = 16
TOP_K = 2

N_STREAMS = 2
BLOCK_T = 1024
HALF = BLOCK_T // N_STREAMS

_NT = (((1,), (1,)), ((), ()))


def _route_stream(xb, w_t, b_col, eps_blk):
    both_t = lax.dot_general(w_t, xb, _NT,
                             preferred_element_type=jnp.float32)
    both_t = both_t + b_col
    logits_t = both_t[:NUM_EXPERTS, :]
    nlogits_t = both_t[NUM_EXPERTS:, :]
    noisy = logits_t + eps_blk * jax.nn.softplus(nlogits_t)

    iota = lax.broadcasted_iota(jnp.int32, noisy.shape, 0)
    m1 = jnp.max(noisy, axis=0, keepdims=True)
    i1 = jnp.min(jnp.where(noisy == m1, iota, NUM_EXPERTS), axis=0,
                 keepdims=True)
    masked = jnp.where(iota == i1, -jnp.inf, noisy)
    m2 = jnp.max(masked, axis=0, keepdims=True)
    i2 = jnp.min(jnp.where(masked == m2, iota, NUM_EXPERTS), axis=0,
                 keepdims=True)
    keep = (iota == i1) | (iota == i2)
    z = jnp.where(keep, jnp.exp(noisy - m1), 0.0)
    out = z / jnp.sum(z, axis=0, keepdims=True)
    idx = jnp.concatenate([i1, i2], axis=0)
    return out, idx


def _router_body(*refs):
    x_refs = refs[:N_STREAMS]
    wr_ref, wn_ref, br_ref, bn_ref, eps_ref, out_ref, idx_ref = \
        refs[N_STREAMS:]
    w_t = jnp.concatenate([wr_ref[...], wn_ref[...]], axis=0)
    b_col = jnp.concatenate([br_ref[...], bn_ref[...]], axis=1).T
    for s in range(N_STREAMS):
        lo = s * HALF
        out_s, idx_s = _route_stream(x_refs[s][...], w_t, b_col,
                                     eps_ref[:, pl.ds(lo, HALF)])
        out_ref[:, pl.ds(lo, HALF)] = out_s
        idx_ref[:, pl.ds(lo, HALF)] = idx_s


def _x_spec(s):
    return pl.BlockSpec((HALF, D_MODEL), lambda i: (N_STREAMS * i + s, 0))


def kernel(x, W_route, b_route, W_noise, b_noise, eps):
    n_blocks = N_TOKENS // BLOCK_T
    wr_t = W_route.T
    wn_t = W_noise.T
    br = b_route.reshape(1, NUM_EXPERTS)
    bn = b_noise.reshape(1, NUM_EXPERTS)
    eps_t = eps.T
    out_shapes = (
        jax.ShapeDtypeStruct((NUM_EXPERTS, N_TOKENS), jnp.float32),
        jax.ShapeDtypeStruct((TOP_K, N_TOKENS), jnp.int32),
    )
    out_t, idx_t = pl.pallas_call(
        _router_body,
        grid=(n_blocks,),
        in_specs=[_x_spec(s) for s in range(N_STREAMS)] + [
            pl.BlockSpec((NUM_EXPERTS, D_MODEL), lambda i: (0, 0)),
            pl.BlockSpec((NUM_EXPERTS, D_MODEL), lambda i: (0, 0)),
            pl.BlockSpec((1, NUM_EXPERTS), lambda i: (0, 0)),
            pl.BlockSpec((1, NUM_EXPERTS), lambda i: (0, 0)),
            pl.BlockSpec((NUM_EXPERTS, BLOCK_T), lambda i: (0, i)),
        ],
        out_specs=(
            pl.BlockSpec((NUM_EXPERTS, BLOCK_T), lambda i: (0, i)),
            pl.BlockSpec((TOP_K, BLOCK_T), lambda i: (0, i)),
        ),
        out_shape=out_shapes,
    )(*([x] * N_STREAMS), wr_t, wn_t, br, bn, eps_t)
    return (out_t.T, idx_t.T)
